# baseline (speedup 1.0000x reference)
import jax
import jax.numpy as jnp
import numpy as np
from jax.experimental import pallas as pl
from jax.experimental.pallas import tpu as pltpu

ALPHA = 4


def _make_body(idx):
    def body(in_ref, slow_ref, fast_ref):
        fast_ref[...] = in_ref[...]
        for s, i in enumerate(idx):
            slow_ref[:, s] = in_ref[:, i]

    return body


def kernel(frames):
    B, C, T, H, W = frames.shape
    S = T // ALPHA
    idx = [int(v) for v in np.linspace(0.0, T - 1, S, dtype=np.float32).astype(np.int32)]
    D = H * W
    L = 128
    M = D // L
    x = frames.reshape(B * C, T, M, L)
    slow, fast = pl.pallas_call(
        _make_body(idx),
        grid=(B * C,),
        in_specs=[pl.BlockSpec((1, T, M, L), lambda bc: (bc, 0, 0, 0))],
        out_specs=[
            pl.BlockSpec((1, S, M, L), lambda bc: (bc, 0, 0, 0)),
            pl.BlockSpec((1, T, M, L), lambda bc: (bc, 0, 0, 0)),
        ],
        out_shape=[
            jax.ShapeDtypeStruct((B * C, S, M, L), frames.dtype),
            jax.ShapeDtypeStruct((B * C, T, M, L), frames.dtype),
        ],
    )(x)
    return slow.reshape(B, C, S, H, W), fast.reshape(B, C, T, H, W)
